# baseline (speedup 1.0000x reference)
import functools

import jax
import jax.numpy as jnp
from jax import lax
from jax.experimental import pallas as pl
from jax.experimental.pallas import tpu as pltpu
from jax.experimental.pallas import tpu_sc as plsc

EMB = 64
EMBW = EMB // 2
LANES = 16
NC, NS = 2, 16
NW = NC * NS
CB = 64
NBUF = 2

VOCAB = 1000000
TR_BLK = 4096
NBT = -(-VOCAB // TR_BLK)
TR_GRID = -(-NBT // 4)
P_ROWS = TR_BLK * TR_GRID


def _pack_pair(lo, hi):
    def rnd(x):
        xi = lax.bitcast_convert_type(x, jnp.uint32)
        return (xi + 0x7FFF + ((xi >> 16) & 1)) >> 16

    word = rnd(lo) | (rnd(hi) << 16)
    return lax.bitcast_convert_type(word, jnp.float32)


def _tr_body(xu0, xu1, xu2, xu3, xv0, xv1, xv2, xv3, ou_ref, ov_ref):
    for xa, xb, xc, xd, o_ref in ((xu0, xu1, xu2, xu3, ou_ref),
                                  (xv0, xv1, xv2, xv3, ov_ref)):
        y1 = jnp.concatenate([xa[...], xb[...]], axis=0).T
        y2 = jnp.concatenate([xc[...], xd[...]], axis=0).T
        o_ref[:, 0:32] = _pack_pair(y1[:, 0:32], y1[:, 32:64])
        o_ref[:, 32:64] = _pack_pair(y1[:, 64:96], y1[:, 96:128])
        o_ref[:, 64:96] = _pack_pair(y2[:, 0:32], y2[:, 32:64])
        o_ref[:, 96:128] = _pack_pair(y2[:, 64:96], y2[:, 96:128])


def _repack_tables(emb_u, emb_v):
    u_t, v_t = emb_u.T, emb_v.T
    spec = [pl.BlockSpec((64, TR_BLK), ix) for ix in (
        lambda i: (0, i),
        lambda i: (0, NBT - 1 - i),
        lambda i: (0, TR_GRID + i),
        lambda i: (0, NBT - 1 - TR_GRID - i),
    )]
    out = jax.ShapeDtypeStruct((P_ROWS, 128), jnp.float32)
    pu, pv = pl.pallas_call(
        _tr_body,
        out_shape=(out, out),
        grid=(TR_GRID,),
        in_specs=spec + spec,
        out_specs=(pl.BlockSpec((TR_BLK, 128), lambda i: (i, 0)),) * 2,
    )(u_t, u_t, u_t, u_t, v_t, v_t, v_t, v_t)
    shp = (4 * P_ROWS, EMBW)
    return pu.reshape(shp), pv.reshape(shp)


def _remap_idx(w):
    blk = w // TR_BLK
    off = w % TR_BLK
    i_q = jnp.where(
        blk < TR_GRID, blk,
        jnp.where(blk < 2 * TR_GRID, blk - TR_GRID,
                  jnp.where(blk <= NBT - 1 - TR_GRID, NBT - 1 - TR_GRID - blk,
                            NBT - 1 - blk)))
    q = jnp.where(
        blk < TR_GRID, 0,
        jnp.where(blk < 2 * TR_GRID, 2,
                  jnp.where(blk <= NBT - 1 - TR_GRID, 3, 1)))
    return 4 * (i_q * TR_BLK + off) + q


def _sc_scores_kernel(B, K):
    KP1 = K + 1
    BPW = B // NW
    NCHUNK = BPW // CB
    mesh = plsc.VectorSubcoreMesh(core_axis_name="c", subcore_axis_name="s")

    @functools.partial(
        pl.kernel,
        out_type=jax.ShapeDtypeStruct((NW, NCHUNK, KP1 * CB), jnp.float32),
        mesh=mesh,
        scratch_types=[
            pltpu.VMEM((KP1, BPW), jnp.int32),
            pltpu.VMEM((BPW,), jnp.int32),
            pltpu.VMEM((NBUF, CB, EMBW), jnp.float32),
            pltpu.VMEM((NBUF, KP1 * CB, EMBW), jnp.float32),
            pltpu.VMEM((NBUF, KP1 * CB), jnp.float32),
            pltpu.SemaphoreType.DMA,
            pltpu.SemaphoreType.DMA,
        ],
        compiler_params=pltpu.CompilerParams(
            needs_layout_passes=False, use_tc_tiling_on_sc=False),
    )
    def sc_kernel(uidx_hbm, cidx_hbm, emb_u, emb_v, out_hbm,
                  uidx_v, cidx_v, crows_v, urows_v, scores_v, *sems):
        w = lax.axis_index("s") * NC + lax.axis_index("c")
        pltpu.sync_copy(uidx_hbm.at[w], uidx_v)
        pltpu.sync_copy(cidx_hbm.at[w], cidx_v)

        lanes = lax.iota(jnp.int32, LANES)

        def issue(ch):
            buf = ch % NBUF
            base = ch * CB
            copies = [pltpu.async_copy(
                emb_v.at[cidx_v.at[pl.ds(base, CB)]], crows_v.at[buf],
                sems[buf])]
            for k in range(KP1):
                copies.append(pltpu.async_copy(
                    emb_u.at[uidx_v.at[k, pl.ds(base, CB)]],
                    urows_v.at[buf, pl.ds(k * CB, CB)], sems[buf]))
            return copies

        def unpack(ref, rows, jskew):
            word = plsc.load_gather(ref, [rows, jskew])
            return plsc.unpack(plsc.bitcast(word, jnp.bfloat16),
                               format=plsc.PackFormat.INTERLEAVED)

        def compute(ch):
            buf = ch % NBUF
            crows = crows_v.at[buf]
            urows = urows_v.at[buf]
            for g in range(CB // LANES):
                blrow = g * LANES + lanes

                def j_body(j, accs):
                    jskew = (jnp.full((LANES,), j, jnp.int32) + lanes) & (
                        EMBW - 1)
                    ca, cb = unpack(crows, blrow, jskew)
                    new = []
                    for k in range(KP1):
                        ua, ub = unpack(urows, blrow + (k * CB), jskew)
                        new.append(accs[k] + ua * ca + ub * cb)
                    return tuple(new)

                accs = lax.fori_loop(
                    0, EMBW, j_body,
                    tuple(jnp.zeros((LANES,), jnp.float32)
                          for _ in range(KP1)))
                for k in range(KP1):
                    scores_v[buf, pl.ds(k * CB + g * LANES, LANES)] = (
                        accs[k] if k == 0 else -accs[k])

            pltpu.sync_copy(scores_v.at[buf], out_hbm.at[w, ch])

        pending = {0: issue(0)}
        for ch in range(NCHUNK):
            for c in pending.pop(ch):
                c.wait()
            if ch + 1 < NCHUNK:
                pending[ch + 1] = issue(ch + 1)
            compute(ch)

    return sc_kernel


def _tc_loss_body(s_ref, o_ref):
    x = s_ref[...]
    ls = jnp.minimum(x, 0.0) - jnp.log(1.0 + jnp.exp(-jnp.abs(x)))
    o_ref[0, 0] = jnp.sum(ls)


def kernel(center_words, target_words, negative_words, embedding_u, embedding_v):
    B, K = negative_words.shape
    KP1 = K + 1
    BPW = B // NW

    lin_u, lin_v = _repack_tables(embedding_u, embedding_v)

    u_idx = jnp.concatenate([target_words, negative_words], axis=1)
    u_idx = _remap_idx(u_idx).reshape(NW, BPW, KP1).transpose(0, 2, 1)
    c_idx = _remap_idx(center_words).reshape(NW, BPW)

    scores = _sc_scores_kernel(B, K)(u_idx, c_idx, lin_u, lin_v)
    total = B * KP1
    scores2d = scores.reshape(total // 128, 128)

    loss_sum = pl.pallas_call(
        _tc_loss_body,
        out_shape=jax.ShapeDtypeStruct((1, 1), jnp.float32),
        in_specs=[pl.BlockSpec(memory_space=pltpu.VMEM)],
        out_specs=pl.BlockSpec(memory_space=pltpu.SMEM),
    )(scores2d)
    return -loss_sum[0, 0] / B


# speedup vs baseline: 1.6177x; 1.6177x over previous
import functools

import jax
import jax.numpy as jnp
from jax import lax
from jax.experimental import pallas as pl
from jax.experimental.pallas import tpu as pltpu
from jax.experimental.pallas import tpu_sc as plsc

EMB = 64
EMBW = EMB // 2
LANES = 16
NC, NS = 2, 16
NW = NC * NS
CB = 64
NBUF = 2

VOCAB = 1000000
TR_BLK = 4096
NBT = -(-VOCAB // TR_BLK)
TR_GRID = -(-NBT // 4)
P_ROWS = TR_BLK * TR_GRID


def _pack_pair(lo, hi):
    li = lax.bitcast_convert_type(lo, jnp.uint32)
    hi_ = lax.bitcast_convert_type(hi, jnp.uint32)
    word = ((li + 0x8000) >> 16) | ((hi_ + 0x8000) & jnp.uint32(0xFFFF0000))
    return lax.bitcast_convert_type(word, jnp.float32)


def _tr_body(xu0, xu1, xu2, xu3, xv0, xv1, xv2, xv3, ou_ref, ov_ref):
    for xa, xb, xc, xd, o_ref in ((xu0, xu1, xu2, xu3, ou_ref),
                                  (xv0, xv1, xv2, xv3, ov_ref)):
        parts = [_pack_pair(x[0:32, :], x[32:64, :])
                 for x in (xa[...], xb[...], xc[...], xd[...])]
        z = jnp.concatenate(parts, axis=0)
        o_ref[...] = z.T


def _repack_tables(emb_u, emb_v):
    u_t, v_t = emb_u.T, emb_v.T
    spec = [pl.BlockSpec((64, TR_BLK), ix) for ix in (
        lambda i: (0, i),
        lambda i: (0, NBT - 1 - i),
        lambda i: (0, TR_GRID + i),
        lambda i: (0, NBT - 1 - TR_GRID - i),
    )]
    out = jax.ShapeDtypeStruct((P_ROWS, 128), jnp.float32)
    pu, pv = pl.pallas_call(
        _tr_body,
        out_shape=(out, out),
        grid=(TR_GRID,),
        in_specs=spec + spec,
        out_specs=(pl.BlockSpec((TR_BLK, 128), lambda i: (i, 0)),) * 2,
    )(u_t, u_t, u_t, u_t, v_t, v_t, v_t, v_t)
    shp = (4 * P_ROWS, EMBW)
    return pu.reshape(shp), pv.reshape(shp)


def _remap_idx(w):
    blk = w // TR_BLK
    off = w % TR_BLK
    i_q = jnp.where(
        blk < TR_GRID, blk,
        jnp.where(blk < 2 * TR_GRID, blk - TR_GRID,
                  jnp.where(blk <= NBT - 1 - TR_GRID, NBT - 1 - TR_GRID - blk,
                            NBT - 1 - blk)))
    q = jnp.where(
        blk < TR_GRID, 0,
        jnp.where(blk < 2 * TR_GRID, 2,
                  jnp.where(blk <= NBT - 1 - TR_GRID, 3, 1)))
    return 4 * (i_q * TR_BLK + off) + q


def _sc_scores_kernel(B, K):
    KP1 = K + 1
    BPW = B // NW
    NCHUNK = BPW // CB
    mesh = plsc.VectorSubcoreMesh(core_axis_name="c", subcore_axis_name="s")

    @functools.partial(
        pl.kernel,
        out_type=jax.ShapeDtypeStruct((NW, NCHUNK, KP1 * CB), jnp.float32),
        mesh=mesh,
        scratch_types=[
            pltpu.VMEM((KP1, BPW), jnp.int32),
            pltpu.VMEM((BPW,), jnp.int32),
            pltpu.VMEM((NBUF, CB, EMBW), jnp.float32),
            pltpu.VMEM((NBUF, KP1 * CB, EMBW), jnp.float32),
            pltpu.VMEM((NBUF, KP1 * CB), jnp.float32),
            pltpu.SemaphoreType.DMA,
            pltpu.SemaphoreType.DMA,
        ],
        compiler_params=pltpu.CompilerParams(
            needs_layout_passes=False, use_tc_tiling_on_sc=False),
    )
    def sc_kernel(uidx_hbm, cidx_hbm, emb_u, emb_v, out_hbm,
                  uidx_v, cidx_v, crows_v, urows_v, scores_v, *sems):
        w = lax.axis_index("s") * NC + lax.axis_index("c")
        pltpu.sync_copy(uidx_hbm.at[w], uidx_v)
        pltpu.sync_copy(cidx_hbm.at[w], cidx_v)

        lanes = lax.iota(jnp.int32, LANES)

        def issue(ch):
            buf = ch % NBUF
            base = ch * CB
            copies = [pltpu.async_copy(
                emb_v.at[cidx_v.at[pl.ds(base, CB)]], crows_v.at[buf],
                sems[buf])]
            for k in range(KP1):
                copies.append(pltpu.async_copy(
                    emb_u.at[uidx_v.at[k, pl.ds(base, CB)]],
                    urows_v.at[buf, pl.ds(k * CB, CB)], sems[buf]))
            return copies

        def unpack(ref, rows, jskew):
            word = plsc.load_gather(ref, [rows, jskew])
            return plsc.unpack(plsc.bitcast(word, jnp.bfloat16),
                               format=plsc.PackFormat.INTERLEAVED)

        def compute(ch):
            buf = ch % NBUF
            crows = crows_v.at[buf]
            urows = urows_v.at[buf]
            for g in range(CB // LANES):
                blrow = g * LANES + lanes

                def j_body(j, accs):
                    jskew = (jnp.full((LANES,), j, jnp.int32) + lanes) & (
                        EMBW - 1)
                    ca, cb = unpack(crows, blrow, jskew)
                    new = []
                    for k in range(KP1):
                        ua, ub = unpack(urows, blrow + (k * CB), jskew)
                        new.append(accs[k] + ua * ca + ub * cb)
                    return tuple(new)

                accs = lax.fori_loop(
                    0, EMBW, j_body,
                    tuple(jnp.zeros((LANES,), jnp.float32)
                          for _ in range(KP1)))
                for k in range(KP1):
                    scores_v[buf, pl.ds(k * CB + g * LANES, LANES)] = (
                        accs[k] if k == 0 else -accs[k])

            pltpu.sync_copy(scores_v.at[buf], out_hbm.at[w, ch])

        pending = {0: issue(0)}
        for ch in range(NCHUNK):
            for c in pending.pop(ch):
                c.wait()
            if ch + 1 < NCHUNK:
                pending[ch + 1] = issue(ch + 1)
            compute(ch)

    return sc_kernel


def _tc_loss_body(s_ref, o_ref):
    x = s_ref[...]
    ls = jnp.minimum(x, 0.0) - jnp.log(1.0 + jnp.exp(-jnp.abs(x)))
    o_ref[0, 0] = jnp.sum(ls)


def kernel(center_words, target_words, negative_words, embedding_u, embedding_v):
    B, K = negative_words.shape
    KP1 = K + 1
    BPW = B // NW

    lin_u, lin_v = _repack_tables(embedding_u, embedding_v)

    u_idx = jnp.concatenate([target_words, negative_words], axis=1)
    u_idx = _remap_idx(u_idx).reshape(NW, BPW, KP1).transpose(0, 2, 1)
    c_idx = _remap_idx(center_words).reshape(NW, BPW)

    scores = _sc_scores_kernel(B, K)(u_idx, c_idx, lin_u, lin_v)
    total = B * KP1
    scores2d = scores.reshape(total // 128, 128)

    loss_sum = pl.pallas_call(
        _tc_loss_body,
        out_shape=jax.ShapeDtypeStruct((1, 1), jnp.float32),
        in_specs=[pl.BlockSpec(memory_space=pltpu.VMEM)],
        out_specs=pl.BlockSpec(memory_space=pltpu.SMEM),
    )(scores2d)
    return -loss_sum[0, 0] / B
